# initial kernel scaffold (unmeasured)
import jax
import jax.numpy as jnp
from jax import lax
from jax.experimental import pallas as pl
from jax.experimental.pallas import tpu as pltpu

N_DEV = 16
N_EXP = 32
EPD = 2
CAP = 64


def _a2a_moe_a2a(send_buf, w1, w2):
    n_dev, epd, cap, d = send_buf.shape
    f = w1.shape[2]

    def body(send_ref, w1_ref, w2_ref, out_ref, recv_ref, mid_ref, ret_ref,
             s1, r1, s2, r2):
        me = lax.axis_index("i")

        p1 = []
        for o in range(1, n_dev):
            rdma = pltpu.make_async_remote_copy(
                src_ref=send_ref.at[o],
                dst_ref=recv_ref.at[o],
                send_sem=s1.at[o],
                recv_sem=r1.at[o],
                device_id=(lax.rem(me + o, n_dev),),
                device_id_type=pl.DeviceIdType.MESH,
            )
            rdma.start()
            p1.append(rdma)
        recv_ref[0] = send_ref[0]
        for rdma in p1:
            rdma.wait()

        for j in range(epd):
            a = recv_ref[:, j].reshape(n_dev * cap, d)
            h = jnp.maximum(
                jnp.dot(a, w1_ref[j], preferred_element_type=jnp.float32), 0.0
            ).astype(jnp.bfloat16)
            r = jnp.dot(h, w2_ref[j], preferred_element_type=jnp.float32)
            mid_ref[:, j] = r.astype(jnp.bfloat16).reshape(n_dev, cap, d)

        p2 = []
        for o in range(1, n_dev):
            rdma = pltpu.make_async_remote_copy(
                src_ref=mid_ref.at[o],
                dst_ref=ret_ref.at[o],
                send_sem=s2.at[o],
                recv_sem=r2.at[o],
                device_id=(lax.rem(me - o + n_dev, n_dev),),
                device_id_type=pl.DeviceIdType.MESH,
            )
            rdma.start()
            p2.append(rdma)
        ret_ref[0] = mid_ref[0]
        for rdma in p2:
            rdma.wait()

        out_ref[...] = ret_ref[...]

    return pl.pallas_call(
        body,
        out_shape=jax.ShapeDtypeStruct((n_dev, epd, cap, d), jnp.bfloat16),
        in_specs=[pl.BlockSpec(memory_space=pltpu.VMEM)] * 3,
        out_specs=pl.BlockSpec(memory_space=pltpu.VMEM),
        scratch_shapes=[
            pltpu.VMEM((n_dev, epd, cap, d), jnp.bfloat16),
            pltpu.VMEM((n_dev, epd, cap, d), jnp.bfloat16),
            pltpu.VMEM((n_dev, epd, cap, d), jnp.bfloat16),
            pltpu.SemaphoreType.DMA((n_dev,)),
            pltpu.SemaphoreType.DMA((n_dev,)),
            pltpu.SemaphoreType.DMA((n_dev,)),
            pltpu.SemaphoreType.DMA((n_dev,)),
        ],
        compiler_params=pltpu.CompilerParams(collective_id=0),
    )(send_buf, w1, w2)


def kernel(x, assign, W1, W2):
    t, d = x.shape
    me = lax.axis_index("i")

    e = assign.astype(jnp.int32)
    off = (e // EPD - me) % N_DEV
    slot = e % EPD
    onehot = (e[:, None] == jnp.arange(N_EXP, dtype=jnp.int32)[None, :])
    rank = (
        jnp.take_along_axis(
            jnp.cumsum(onehot.astype(jnp.int32), axis=0), e[:, None], axis=1
        )[:, 0]
        - 1
    )

    send_buf = jnp.zeros((N_DEV, EPD, CAP, d), jnp.bfloat16)
    send_buf = send_buf.at[off, slot, rank].set(
        x.astype(jnp.bfloat16), mode="drop"
    )

    ret = _a2a_moe_a2a(
        send_buf, W1.astype(jnp.bfloat16), W2.astype(jnp.bfloat16)
    )

    return ret[off, slot, rank].astype(jnp.float32)


# baseline (device time: 192308 ns/iter reference)
import jax
import jax.numpy as jnp
from jax import lax
from jax.experimental import pallas as pl
from jax.experimental.pallas import tpu as pltpu

N_DEV = 16
N_EXP = 32
EPD = 2
CAP = 64


def _a2a_moe_a2a(send_buf, w1, w2):
    n_dev, epd, cap, d = send_buf.shape
    f = w1.shape[2]

    def body(send_ref, w1_ref, w2_ref, out_ref, recv_ref, mid_ref, ret_ref,
             s1, r1, s2, r2):
        me = lax.axis_index("i")

        p1 = []
        for o in range(1, n_dev):
            rdma = pltpu.make_async_remote_copy(
                src_ref=send_ref.at[o],
                dst_ref=recv_ref.at[o],
                send_sem=s1.at[o],
                recv_sem=r1.at[o],
                device_id=(lax.rem(me + o, n_dev),),
                device_id_type=pl.DeviceIdType.MESH,
            )
            rdma.start()
            p1.append(rdma)
        recv_ref[0] = send_ref[0]
        for rdma in p1:
            rdma.wait()

        for j in range(epd):
            a = recv_ref[:, j].reshape(n_dev * cap, d)
            h = jnp.maximum(
                jnp.dot(a, w1_ref[j], preferred_element_type=jnp.float32), 0.0
            ).astype(jnp.bfloat16)
            r = jnp.dot(h, w2_ref[j], preferred_element_type=jnp.float32)
            mid_ref[:, j] = r.astype(jnp.bfloat16).reshape(n_dev, cap, d)

        p2 = []
        for o in range(1, n_dev):
            rdma = pltpu.make_async_remote_copy(
                src_ref=mid_ref.at[o],
                dst_ref=ret_ref.at[o],
                send_sem=s2.at[o],
                recv_sem=r2.at[o],
                device_id=(lax.rem(me - o + n_dev, n_dev),),
                device_id_type=pl.DeviceIdType.MESH,
            )
            rdma.start()
            p2.append(rdma)
        ret_ref[0] = mid_ref[0]
        for rdma in p2:
            rdma.wait()

        out_ref[...] = ret_ref[...]

    return pl.pallas_call(
        body,
        out_shape=jax.ShapeDtypeStruct((n_dev, epd, cap, d), jnp.bfloat16),
        in_specs=[pl.BlockSpec(memory_space=pltpu.VMEM)] * 3,
        out_specs=pl.BlockSpec(memory_space=pltpu.VMEM),
        scratch_shapes=[
            pltpu.VMEM((n_dev, epd, cap, d), jnp.bfloat16),
            pltpu.VMEM((n_dev, epd, cap, d), jnp.bfloat16),
            pltpu.VMEM((n_dev, epd, cap, d), jnp.bfloat16),
            pltpu.SemaphoreType.DMA((n_dev,)),
            pltpu.SemaphoreType.DMA((n_dev,)),
            pltpu.SemaphoreType.DMA((n_dev,)),
            pltpu.SemaphoreType.DMA((n_dev,)),
        ],
    )(send_buf, w1, w2)


def kernel(x, assign, W1, W2):
    t, d = x.shape
    me = lax.axis_index("i")

    e = assign.astype(jnp.int32)
    off = (e // EPD - me) % N_DEV
    slot = e % EPD
    onehot = (e[:, None] == jnp.arange(N_EXP, dtype=jnp.int32)[None, :])
    rank = (
        jnp.take_along_axis(
            jnp.cumsum(onehot.astype(jnp.int32), axis=0), e[:, None], axis=1
        )[:, 0]
        - 1
    )

    send_buf = jnp.zeros((N_DEV, EPD, CAP, d), jnp.bfloat16)
    send_buf = send_buf.at[off, slot, rank].set(
        x.astype(jnp.bfloat16), mode="drop"
    )

    ret = _a2a_moe_a2a(
        send_buf, W1.astype(jnp.bfloat16), W2.astype(jnp.bfloat16)
    )

    return ret[off, slot, rank].astype(jnp.float32)


# device time: 182571 ns/iter; 1.0533x vs baseline; 1.0533x over previous
import jax
import jax.numpy as jnp
from jax import lax
from jax.experimental import pallas as pl
from jax.experimental.pallas import tpu as pltpu

N_DEV = 16
N_EXP = 32
EPD = 2
CAP = 64


def _a2a_moe_a2a(send_buf, w1, w2):
    n_dev, epd, cap, d = send_buf.shape
    f = w1.shape[2]

    group = 4

    def body(send_ref, w1_ref, w2_ref, out_ref, recv_ref, mid_ref, ret_ref,
             s1, r1, s2, r2):
        me = lax.axis_index("i")

        p1 = {}
        for o in range(1, n_dev):
            rdma = pltpu.make_async_remote_copy(
                src_ref=send_ref.at[o],
                dst_ref=recv_ref.at[o],
                send_sem=s1.at[o],
                recv_sem=r1.at[o],
                device_id=(lax.rem(me + o, n_dev),),
                device_id_type=pl.DeviceIdType.MESH,
            )
            rdma.start()
            p1[o] = rdma
        recv_ref[0] = send_ref[0]

        p2 = {}
        for g0 in range(0, n_dev, group):
            offs = range(g0, min(g0 + group, n_dev))
            for o in offs:
                if o > 0:
                    p1[o].wait_recv()
            for j in range(epd):
                a = recv_ref[pl.ds(g0, group), j].reshape(group * cap, d)
                h = jnp.maximum(
                    jnp.dot(a, w1_ref[j], preferred_element_type=jnp.float32),
                    0.0,
                ).astype(jnp.bfloat16)
                r = jnp.dot(h, w2_ref[j], preferred_element_type=jnp.float32)
                mid_ref[pl.ds(g0, group), j] = r.astype(jnp.bfloat16).reshape(
                    group, cap, d
                )
            for o in offs:
                if o == 0:
                    ret_ref[0] = mid_ref[0]
                    continue
                rdma = pltpu.make_async_remote_copy(
                    src_ref=mid_ref.at[o],
                    dst_ref=ret_ref.at[o],
                    send_sem=s2.at[o],
                    recv_sem=r2.at[o],
                    device_id=(lax.rem(me - o + n_dev, n_dev),),
                    device_id_type=pl.DeviceIdType.MESH,
                )
                rdma.start()
                p2[o] = rdma

        for o in range(1, n_dev):
            p1[o].wait_send()
            p2[o].wait_send()
            p2[o].wait_recv()

        out_ref[...] = ret_ref[...]

    return pl.pallas_call(
        body,
        out_shape=jax.ShapeDtypeStruct((n_dev, epd, cap, d), jnp.bfloat16),
        in_specs=[pl.BlockSpec(memory_space=pltpu.VMEM)] * 3,
        out_specs=pl.BlockSpec(memory_space=pltpu.VMEM),
        scratch_shapes=[
            pltpu.VMEM((n_dev, epd, cap, d), jnp.bfloat16),
            pltpu.VMEM((n_dev, epd, cap, d), jnp.bfloat16),
            pltpu.VMEM((n_dev, epd, cap, d), jnp.bfloat16),
            pltpu.SemaphoreType.DMA((n_dev,)),
            pltpu.SemaphoreType.DMA((n_dev,)),
            pltpu.SemaphoreType.DMA((n_dev,)),
            pltpu.SemaphoreType.DMA((n_dev,)),
        ],
    )(send_buf, w1, w2)


def kernel(x, assign, W1, W2):
    t, d = x.shape
    me = lax.axis_index("i")

    e = assign.astype(jnp.int32)
    off = (e // EPD - me) % N_DEV
    slot = e % EPD
    onehot = (e[:, None] == jnp.arange(N_EXP, dtype=jnp.int32)[None, :])
    rank = (
        jnp.take_along_axis(
            jnp.cumsum(onehot.astype(jnp.int32), axis=0), e[:, None], axis=1
        )[:, 0]
        - 1
    )

    send_buf = jnp.zeros((N_DEV, EPD, CAP, d), jnp.bfloat16)
    send_buf = send_buf.at[off, slot, rank].set(
        x.astype(jnp.bfloat16), mode="drop"
    )

    ret = _a2a_moe_a2a(
        send_buf, W1.astype(jnp.bfloat16), W2.astype(jnp.bfloat16)
    )

    return ret[off, slot, rank].astype(jnp.float32)


# device time: 182529 ns/iter; 1.0536x vs baseline; 1.0002x over previous
import jax
import jax.numpy as jnp
from jax import lax
from jax.experimental import pallas as pl
from jax.experimental.pallas import tpu as pltpu

N_DEV = 16
N_EXP = 32
EPD = 2
CAP = 64


def _a2a_moe_a2a(send_buf, w1, w2):
    n_dev, epd, cap, d = send_buf.shape
    f = w1.shape[2]

    group = 4

    def body(send_ref, w1_ref, w2_ref, out_ref, recv_ref, mid_ref,
             s1, r1, s2, r2):
        me = lax.axis_index("i")

        p1 = {}

        def issue_round(g0):
            for o in range(g0, min(g0 + group, n_dev)):
                if o == 0:
                    continue
                rdma = pltpu.make_async_remote_copy(
                    src_ref=send_ref.at[o],
                    dst_ref=recv_ref.at[o],
                    send_sem=s1.at[o],
                    recv_sem=r1.at[o],
                    device_id=(lax.rem(me + o, n_dev),),
                    device_id_type=pl.DeviceIdType.MESH,
                )
                rdma.start()
                p1[o] = rdma

        issue_round(0)
        issue_round(group)
        recv_ref[0] = send_ref[0]

        p2 = {}
        for g0 in range(0, n_dev, group):
            offs = range(g0, min(g0 + group, n_dev))
            for o in offs:
                if o > 0:
                    p1[o].wait_recv()
            if g0 + 2 * group < n_dev:
                issue_round(g0 + 2 * group)
            for j in range(epd):
                a = recv_ref[pl.ds(g0, group), j].reshape(group * cap, d)
                h = jnp.maximum(
                    jnp.dot(a, w1_ref[j], preferred_element_type=jnp.float32),
                    0.0,
                ).astype(jnp.bfloat16)
                r = jnp.dot(h, w2_ref[j], preferred_element_type=jnp.float32)
                mid_ref[pl.ds(g0, group), j] = r.astype(jnp.bfloat16).reshape(
                    group, cap, d
                )
            for o in offs:
                if o == 0:
                    out_ref[0] = mid_ref[0]
                    continue
                rdma = pltpu.make_async_remote_copy(
                    src_ref=mid_ref.at[o],
                    dst_ref=out_ref.at[o],
                    send_sem=s2.at[o],
                    recv_sem=r2.at[o],
                    device_id=(lax.rem(me - o + n_dev, n_dev),),
                    device_id_type=pl.DeviceIdType.MESH,
                )
                rdma.start()
                p2[o] = rdma

        for o in range(1, n_dev):
            p1[o].wait_send()
            p2[o].wait_send()
            p2[o].wait_recv()

    return pl.pallas_call(
        body,
        out_shape=jax.ShapeDtypeStruct((n_dev, epd, cap, d), jnp.bfloat16),
        in_specs=[pl.BlockSpec(memory_space=pltpu.VMEM)] * 3,
        out_specs=pl.BlockSpec(memory_space=pltpu.VMEM),
        scratch_shapes=[
            pltpu.VMEM((n_dev, epd, cap, d), jnp.bfloat16),
            pltpu.VMEM((n_dev, epd, cap, d), jnp.bfloat16),
            pltpu.SemaphoreType.DMA((n_dev,)),
            pltpu.SemaphoreType.DMA((n_dev,)),
            pltpu.SemaphoreType.DMA((n_dev,)),
            pltpu.SemaphoreType.DMA((n_dev,)),
        ],
    )(send_buf, w1, w2)


def kernel(x, assign, W1, W2):
    t, d = x.shape
    me = lax.axis_index("i")

    e = assign.astype(jnp.int32)
    off = (e // EPD - me) % N_DEV
    slot = e % EPD
    onehot = (e[:, None] == jnp.arange(N_EXP, dtype=jnp.int32)[None, :])
    rank = (
        jnp.take_along_axis(
            jnp.cumsum(onehot.astype(jnp.int32), axis=0), e[:, None], axis=1
        )[:, 0]
        - 1
    )

    send_buf = jnp.zeros((N_DEV, EPD, CAP, d), jnp.bfloat16)
    send_buf = send_buf.at[off, slot, rank].set(
        x.astype(jnp.bfloat16), mode="drop"
    )

    ret = _a2a_moe_a2a(
        send_buf, W1.astype(jnp.bfloat16), W2.astype(jnp.bfloat16)
    )

    return ret[off, slot, rank].astype(jnp.float32)
